# baseline (speedup 1.0000x reference)
import functools

import jax
import jax.numpy as jnp
from jax import lax
from jax.experimental import pallas as pl
from jax.experimental.pallas import tpu as pltpu
from jax.experimental.pallas import tpu_sc as plsc

V = 1000000
D = 64
B = 16384

NC = 2
NS = 16
L = 16
NW = NC * NS
BPW = B // NW
NCHUNK = 4
CHUNK = BPW // NCHUNK
GPC = CHUNK // L
PD = 2 * D


def _body(e0p_hbm, e1p_hbm, ids0_hbm, ids1_hbm, out_hbm,
          idx0_v, idx1_v, pair0_v, pair1_v, rows0_v, rows1_v, out_v,
          sem_idx, sem0, sem1):
    wid = lax.axis_index("s") * NC + lax.axis_index("c")
    base = wid * BPW

    idx_copies = []
    for c in range(NCHUNK):
        idx_copies.append(pltpu.async_copy(
            ids0_hbm.at[pl.ds(base + c * CHUNK, CHUNK)], idx0_v.at[c],
            sem_idx))
        idx_copies.append(pltpu.async_copy(
            ids1_hbm.at[pl.ds(base + c * CHUNK, CHUNK)], idx1_v.at[c],
            sem_idx))
    for cp in idx_copies:
        cp.wait()

    def pair_body(i, _):
        c = jnp.right_shift(i, 3)
        o = jnp.bitwise_and(i, 7) * L
        sl = pl.ds(i * L, L)
        osl = pl.ds(o, L)
        pair0_v[c, osl] = jnp.right_shift(idx0_v.at[c][osl], 1)
        pair1_v[c, osl] = jnp.right_shift(idx1_v.at[c][osl], 1)
        return _

    lax.fori_loop(0, BPW // L, pair_body, None)

    lane = lax.iota(jnp.int32, L)

    def fire(c):
        buf = c % 2
        return (pltpu.async_copy(e0p_hbm.at[pair0_v.at[c]], rows0_v.at[buf],
                                 sem0),
                pltpu.async_copy(e1p_hbm.at[pair1_v.at[c]], rows1_v.at[buf],
                                 sem1))

    copies = {0: fire(0), 1: fire(1)}

    for c in range(NCHUNK):
        buf = c % 2
        cp0, cp1 = copies.pop(c)
        cp0.wait()
        cp1.wait()

        def group_body(g, _, c=c, buf=buf):
            slot = g * L + lane
            bufv = jnp.full((L,), buf, jnp.int32)
            half0 = jnp.bitwise_and(idx0_v.at[c][pl.ds(g * L, L)], 1) * D
            half1 = jnp.bitwise_and(idx1_v.at[c][pl.ds(g * L, L)], 1) * D
            col = lane
            acc = jnp.zeros((L,), jnp.float32)
            for d in range(D):
                v0 = plsc.load_gather(rows0_v, [bufv, slot, half0 + col])
                v1 = plsc.load_gather(rows1_v, [bufv, slot, half1 + col])
                acc = acc + v0 * v1
                if d + 1 < D:
                    col = col + jnp.where(lane == D - 1 - d, 1 - D, 1)
            out_v[pl.ds(c * CHUNK + g * L, L)] = acc
            return _

        lax.fori_loop(0, GPC, group_body, None)

        if c + 2 < NCHUNK:
            copies[c + 2] = fire(c + 2)

    pltpu.sync_copy(out_v, out_hbm.at[pl.ds(base, BPW)])


@jax.jit
def _run(E0, E1, ids0, ids1):
    mesh = plsc.VectorSubcoreMesh(core_axis_name="c", subcore_axis_name="s")
    kfn = pl.kernel(
        _body,
        out_type=jax.ShapeDtypeStruct((B,), jnp.float32),
        mesh=mesh,
        compiler_params=pltpu.CompilerParams(needs_layout_passes=False),
        scratch_types=[
            pltpu.VMEM((NCHUNK, CHUNK), jnp.int32),
            pltpu.VMEM((NCHUNK, CHUNK), jnp.int32),
            pltpu.VMEM((NCHUNK, CHUNK), jnp.int32),
            pltpu.VMEM((NCHUNK, CHUNK), jnp.int32),
            pltpu.VMEM((2, CHUNK, PD), jnp.float32),
            pltpu.VMEM((2, CHUNK, PD), jnp.float32),
            pltpu.VMEM((BPW,), jnp.float32),
            pltpu.SemaphoreType.DMA,
            pltpu.SemaphoreType.DMA,
            pltpu.SemaphoreType.DMA,
        ],
    )
    return kfn(E0.reshape(V // 2, PD), E1.reshape(V // 2, PD), ids0, ids1)


def kernel(E0, E1, ids0, ids1):
    return _run(E0, E1, ids0, ids1).reshape(B, 1)


# speedup vs baseline: 2.2135x; 2.2135x over previous
import functools

import jax
import jax.numpy as jnp
from jax import lax
from jax.experimental import pallas as pl
from jax.experimental.pallas import tpu as pltpu
from jax.experimental.pallas import tpu_sc as plsc

V = 1000000
D = 64
B = 16384

NC = 2
NS = 16
L = 16
NW = NC * NS
BPW = B // NW
NCHUNK = 32
CHUNK = BPW // NCHUNK
SUB = 8


def _body(e0_hbm, e1_hbm, ids0_hbm, ids1_hbm, out_hbm,
          idx0_v, idx1_v, rows_v0, rows_v1, out_v,
          sem_idx, sem_a, sem_b):
    wid = lax.axis_index("s") * NC + lax.axis_index("c")
    base = wid * BPW

    ci0 = pltpu.async_copy(ids0_hbm.at[pl.ds(base, BPW)], idx0_v, sem_idx)
    ci1 = pltpu.async_copy(ids1_hbm.at[pl.ds(base, BPW)], idx1_v, sem_idx)
    ci0.wait()
    ci1.wait()

    lane = lax.iota(jnp.int32, L)
    sems = [sem_a, sem_b]

    def enqueue_chunk(c, buf):
        idv0 = idx0_v[pl.ds(c * CHUNK, L)]
        idv1 = idx1_v[pl.ds(c * CHUNK, L)]
        for j in range(CHUNK):
            pltpu.async_copy(
                e0_hbm.at[pl.ds(jnp.right_shift(idv0[j], 3), 1)],
                rows_v0.at[buf, pl.ds(j, 1)], sems[buf])
            pltpu.async_copy(
                e1_hbm.at[pl.ds(jnp.right_shift(idv1[j], 3), 1)],
                rows_v1.at[buf, pl.ds(j, 1)], sems[buf])

    def drain_chunk(buf):
        for _j in range(CHUNK):
            pltpu.make_async_copy(e0_hbm.at[pl.ds(0, 1)],
                                  rows_v0.at[buf, pl.ds(0, 1)],
                                  sems[buf]).wait()
            pltpu.make_async_copy(e1_hbm.at[pl.ds(0, 1)],
                                  rows_v1.at[buf, pl.ds(0, 1)],
                                  sems[buf]).wait()

    enqueue_chunk(0, 0)

    def chunk_body(c, _):
        parity = jnp.bitwise_and(c, 1)

        @pl.when(jnp.logical_and(parity == 0, c + 1 < NCHUNK))
        def _():
            enqueue_chunk(c + 1, 1)

        @pl.when(jnp.logical_and(parity == 1, c + 1 < NCHUNK))
        def _():
            enqueue_chunk(c + 1, 0)

        @pl.when(parity == 0)
        def _():
            drain_chunk(0)

        @pl.when(parity == 1)
        def _():
            drain_chunk(1)

        bufv = jnp.broadcast_to(parity, (L,))
        sub0 = jnp.bitwise_and(idx0_v[pl.ds(c * CHUNK, L)], SUB - 1)
        sub1 = jnp.bitwise_and(idx1_v[pl.ds(c * CHUNK, L)], SUB - 1)
        col = lane
        acc = jnp.zeros((L,), jnp.float32)
        for d in range(D):
            v0 = plsc.load_gather(rows_v0, [bufv, lane, sub0, col])
            v1 = plsc.load_gather(rows_v1, [bufv, lane, sub1, col])
            acc = acc + v0 * v1
            if d + 1 < D:
                col = col + jnp.where(lane == D - 1 - d, 1 - D, 1)
        out_v[pl.ds(c * CHUNK, L)] = acc
        return _

    lax.fori_loop(0, NCHUNK, chunk_body, None)

    pltpu.sync_copy(out_v, out_hbm.at[pl.ds(base, BPW)])


@jax.jit
def _run(E0, E1, ids0, ids1):
    mesh = plsc.VectorSubcoreMesh(core_axis_name="c", subcore_axis_name="s")
    kfn = pl.kernel(
        _body,
        out_type=jax.ShapeDtypeStruct((B,), jnp.float32),
        mesh=mesh,
        compiler_params=pltpu.CompilerParams(needs_layout_passes=False),
        scratch_types=[
            pltpu.VMEM((BPW,), jnp.int32),
            pltpu.VMEM((BPW,), jnp.int32),
            pltpu.VMEM((2, CHUNK, SUB, D), jnp.float32),
            pltpu.VMEM((2, CHUNK, SUB, D), jnp.float32),
            pltpu.VMEM((BPW,), jnp.float32),
            pltpu.SemaphoreType.DMA,
            pltpu.SemaphoreType.DMA,
            pltpu.SemaphoreType.DMA,
        ],
    )
    return kfn(E0.reshape(V // SUB, SUB, D), E1.reshape(V // SUB, SUB, D),
               ids0, ids1)


def kernel(E0, E1, ids0, ids1):
    return _run(E0, E1, ids0, ids1).reshape(B, 1)
